# Initial kernel scaffold; baseline (speedup 1.0000x reference)
import jax
import jax.numpy as jnp
from jax.experimental import pallas as pl


def kernel(u, v, negative_v, u_embedding_weight, v_embedding_weight):
    raise NotImplementedError("write your pallas kernel here")


# speedup vs baseline: 5.0136x; 5.0136x over previous
import functools

import jax
import jax.numpy as jnp
from jax import lax
from jax.experimental import pallas as pl
from jax.experimental.pallas import tpu as pltpu
from jax.experimental.pallas import tpu_sc as plsc

VOCAB = 100000
DIM = 128
BATCH = 4096
N_NEG = 20

NC = 2
NS = 16
NW = NC * NS
BPW = BATCH // NW
CHUNK = 4
ROWS = CHUNK * N_NEG
NCHUNK = BPW // CHUNK
LANES = 16
NSUB = DIM // LANES


def _sc_body(u_hbm, v_hbm, negv_hbm, ut_hbm, vt_hbm,
             pos_out, neg_out,
             u_idx, v_idx, neg_idx, u_rows, v_rows, nbuf0, nbuf1,
             pos_s, neg_s, sem_u, sem_v, sem_n0, sem_n1):
    wid = lax.axis_index("s") * NC + lax.axis_index("c")
    base = wid * BPW

    pltpu.sync_copy(u_hbm.at[pl.ds(base, BPW)], u_idx)
    pltpu.sync_copy(v_hbm.at[pl.ds(base, BPW)], v_idx)
    pltpu.sync_copy(negv_hbm.at[pl.ds(wid * NCHUNK, NCHUNK), :], neg_idx)

    cu = pltpu.async_copy(ut_hbm.at[u_idx], u_rows, sem_u)
    cv = pltpu.async_copy(vt_hbm.at[v_idx], v_rows, sem_v)

    nbufs = (nbuf0, nbuf1)
    nsems = (sem_n0, sem_n1)

    def fire(c, d):
        pltpu.async_copy(vt_hbm.at[neg_idx.at[c]], nbufs[d], nsems[d])

    fire(0, 0)
    fire(1, 1)

    cu.wait()
    cv.wait()

    lane_iota = lax.iota(jnp.int32, LANES)
    perms = [lax.rem(lane_iota + sh, jnp.int32(LANES)) for sh in (8, 4, 2, 1)]
    masks = [lane_iota == l for l in range(LANES)]

    def lane_sum(acc):
        for p in perms:
            acc = acc + jnp.take(acc, p)
        return acc

    def pos_body(g, carry):
        vec = None
        for l in range(LANES):
            b = g * LANES + l
            acc = u_rows[b, pl.ds(0, LANES)] * v_rows[b, pl.ds(0, LANES)]
            for s in range(1, NSUB):
                acc = acc + (u_rows[b, pl.ds(s * LANES, LANES)]
                             * v_rows[b, pl.ds(s * LANES, LANES)])
            score = lane_sum(acc)
            vec = score if vec is None else jnp.where(masks[l], score, vec)
        pos_s[pl.ds(g * LANES, LANES)] = vec
        return carry

    lax.fori_loop(0, BPW // LANES, pos_body, 0)

    def pair_body(i, carry):
        for d in range(2):
            c = i * 2 + d
            pltpu.make_async_copy(vt_hbm.at[neg_idx.at[c]],
                                  nbufs[d], nsems[d]).wait()
            us = None
            vec = None
            for r in range(ROWS):
                j, k = divmod(r, N_NEG)
                if k == 0:
                    b = c * CHUNK + j
                    us = [u_rows[b, pl.ds(s * LANES, LANES)]
                          for s in range(NSUB)]
                acc = nbufs[d][r, pl.ds(0, LANES)] * us[0]
                for s in range(1, NSUB):
                    acc = acc + nbufs[d][r, pl.ds(s * LANES, LANES)] * us[s]
                score = lane_sum(acc)
                vec = (score if r % LANES == 0
                       else jnp.where(masks[r % LANES], score, vec))
                if r % LANES == LANES - 1:
                    neg_s[c, pl.ds((r // LANES) * LANES, LANES)] = vec
            nxt = c + 2

            @pl.when(nxt < NCHUNK)
            def _():
                fire(nxt, d)

        return carry

    lax.fori_loop(0, NCHUNK // 2, pair_body, 0)

    pltpu.sync_copy(pos_s, pos_out.at[pl.ds(base, BPW)])
    pltpu.sync_copy(neg_s, neg_out.at[pl.ds(wid * NCHUNK, NCHUNK), :])


_sc_scores = pl.kernel(
    _sc_body,
    out_type=[
        jax.ShapeDtypeStruct((BATCH,), jnp.float32),
        jax.ShapeDtypeStruct((BATCH // CHUNK, ROWS), jnp.float32),
    ],
    mesh=plsc.VectorSubcoreMesh(core_axis_name="c", subcore_axis_name="s"),
    scratch_types=[
        pltpu.VMEM((BPW,), jnp.int32),
        pltpu.VMEM((BPW,), jnp.int32),
        pltpu.VMEM((NCHUNK, ROWS), jnp.int32),
        pltpu.VMEM((BPW, DIM), jnp.float32),
        pltpu.VMEM((BPW, DIM), jnp.float32),
        pltpu.VMEM((ROWS, DIM), jnp.float32),
        pltpu.VMEM((ROWS, DIM), jnp.float32),
        pltpu.VMEM((BPW,), jnp.float32),
        pltpu.VMEM((NCHUNK, ROWS), jnp.float32),
        pltpu.SemaphoreType.DMA,
        pltpu.SemaphoreType.DMA,
        pltpu.SemaphoreType.DMA,
        pltpu.SemaphoreType.DMA,
    ],
)


def _log_sigmoid(x):
    return jnp.minimum(x, 0.0) - jnp.log1p(jnp.exp(-jnp.abs(x)))


def _tc_body(pos_ref, neg_ref, o_ref):
    ls_pos = _log_sigmoid(pos_ref[...])
    ls_neg = _log_sigmoid(-neg_ref[...])
    o_ref[0, 0] = -(jnp.mean(ls_pos) + jnp.mean(ls_neg)) * 0.5


_tc_loss = pl.pallas_call(
    _tc_body,
    out_shape=jax.ShapeDtypeStruct((1, 1), jnp.float32),
    out_specs=pl.BlockSpec(memory_space=pltpu.SMEM),
)


@jax.jit
def kernel(u, v, negative_v, u_embedding_weight, v_embedding_weight):
    neg2d = negative_v.reshape(BATCH // CHUNK, ROWS)
    pos_sc, neg_sc = _sc_scores(u, v, neg2d,
                                u_embedding_weight, v_embedding_weight)
    loss = _tc_loss(pos_sc.reshape(BATCH // DIM, DIM),
                    neg_sc.reshape(BATCH * N_NEG // DIM, DIM))
    return loss[0, 0]
